# baseline (speedup 1.0000x reference)
import functools

import jax
import jax.numpy as jnp
from jax import lax
from jax.experimental import pallas as pl
from jax.experimental.pallas import tpu as pltpu
from jax.experimental.pallas import tpu_sc as plsc

_NC = 2
_NS = 16
_B = 128
_NBT = 80
_PAD = 512


def _sc_mesh():
    return plsc.VectorSubcoreMesh(core_axis_name="c", subcore_axis_name="s")


def _round256(n):
    return (n + 255) & ~255


@functools.partial(jax.jit, static_argnames=("n",))
def _sc_degrees(ei2, ones_h, zeros_h, n):
    nb_tot = ei2.shape[0]
    nb = nb_tot // (_NC * _NS)
    ra = (n // _NS) & ~7
    tail = n - ra * _NS

    @functools.partial(
        pl.kernel,
        mesh=_sc_mesh(),
        out_type=jax.ShapeDtypeStruct((2, _NC, _round256(n + _PAD)),
                                      jnp.float32),
        scratch_types=[
            pltpu.VMEM((nb, 128), jnp.int32),
            pltpu.VMEM((2, 128), jnp.int32),
            pltpu.VMEM((128,), jnp.float32),
            pltpu.VMEM_SHARED((_round256(n + _PAD),), jnp.float32),
            pltpu.VMEM_SHARED((_round256(n + _PAD),), jnp.float32),
        ],
    )
    def hist(ei_h, ones_hbm, zeros_hbm, out_h, ei_v, stg, ones_v,
             acc_s, acc_d):
        core = lax.axis_index("c")
        sub = lax.axis_index("s")
        row0 = (core * _NS + sub) * nb
        pltpu.sync_copy(ei_h.at[pl.ds(row0, nb)], ei_v)
        pltpu.sync_copy(ones_hbm, ones_v)

        @pl.when(sub == 0)
        def _():
            pltpu.sync_copy(zeros_hbm, acc_s)
            pltpu.sync_copy(zeros_hbm, acc_d)

        plsc.subcore_barrier()

        def body(j, carry):
            for d in range(2):
                for q in range(4):
                    v = ei_v[j, pl.ds(d * 64 + q * 16, 16)]
                    stg[d, pl.ds(q * 32, 16)] = v & 0xFFFF
                    stg[d, pl.ds(q * 32 + 16, 16)] = (
                        lax.shift_right_logical(v, 16))
            pltpu.sync_copy(ones_v, acc_s.at[stg.at[0]], add=True)
            pltpu.sync_copy(ones_v, acc_d.at[stg.at[1]], add=True)
            return carry

        lax.fori_loop(0, nb, body, 0)
        plsc.subcore_barrier()

        @pl.when(sub == 0)
        def _():
            pltpu.sync_copy(acc_s, out_h.at[0].at[core])
            pltpu.sync_copy(acc_d, out_h.at[1].at[core])

    return hist(ei2, ones_h, zeros_h)


@functools.partial(jax.jit, static_argnames=("n", "fc", "c_chunks"))
def _sc_msgpass(msg, ei2, zeros_h, n, fc, c_chunks):
    nb_tot = ei2.shape[0]
    nb = nb_tot // (_NC * _NS)
    ra = (n // _NS) & ~7
    tail = n - ra * _NS

    @functools.partial(
        pl.kernel,
        mesh=_sc_mesh(),
        out_type=jax.ShapeDtypeStruct((c_chunks, _NC, n, fc), jnp.float32),
        scratch_types=[
            pltpu.VMEM((nb, 128), jnp.int32),
            pltpu.VMEM((2, 2, _B), jnp.int32),
            pltpu.VMEM((2, _B, fc), jnp.float32),
            pltpu.VMEM_SHARED((n + _PAD, fc), jnp.float32),
            pltpu.SemaphoreType.DMA,
            pltpu.SemaphoreType.DMA,
        ],
    )
    def msgpass(msg_h, ei_h, zeros_hbm, out_h, ei_v, stg, rows, acc,
                sem0, sem1):
        core = lax.axis_index("c")
        sub = lax.axis_index("s")
        row0 = (core * _NS + sub) * nb
        pltpu.sync_copy(ei_h.at[pl.ds(row0, nb)], ei_v)
        sl = pl.ds(sub * ra, ra)
        tl = pl.ds(ra * _NS, tail)
        sems = (sem0, sem1)

        def unpack(j, b):
            for d in range(2):
                for q in range(4):
                    v = ei_v[j, pl.ds(d * 64 + q * 16, 16)]
                    stg[b, d, pl.ds(q * 32, 16)] = v & 0xFFFF
                    stg[b, d, pl.ds(q * 32 + 16, 16)] = (
                        lax.shift_right_logical(v, 16))

        for c in range(c_chunks):
            m = msg_h.at[c]
            pltpu.sync_copy(zeros_hbm.at[pl.ds(0, ra)], acc.at[sl])

            @pl.when(sub == 0)
            def _():
                pltpu.sync_copy(zeros_hbm.at[pl.ds(0, tail)], acc.at[tl])

            plsc.subcore_barrier()

            unpack(0, 0)
            pltpu.async_copy(m.at[stg.at[0, 0]], rows.at[0], sems[0])

            def body(i, carry):
                j = 2 * i
                unpack(j + 1, 1)
                pltpu.async_copy(m.at[stg.at[1, 0]], rows.at[1], sems[1])
                pltpu.make_async_copy(m.at[stg.at[0, 0]], rows.at[0],
                                      sems[0]).wait()
                pltpu.sync_copy(rows.at[0], acc.at[stg.at[0, 1]], add=True)

                @pl.when(i < nb // 2 - 1)
                def _():
                    unpack(j + 2, 0)
                    pltpu.async_copy(m.at[stg.at[0, 0]], rows.at[0], sems[0])

                pltpu.make_async_copy(m.at[stg.at[1, 0]], rows.at[1],
                                      sems[1]).wait()
                pltpu.sync_copy(rows.at[1], acc.at[stg.at[1, 1]], add=True)
                return carry

            lax.fori_loop(0, nb // 2, body, 0)
            plsc.subcore_barrier()
            pltpu.sync_copy(acc.at[sl], out_h.at[c].at[core].at[sl])

            @pl.when(sub == 0)
            def _():
                pltpu.sync_copy(acc.at[tl], out_h.at[c].at[core].at[tl])

            plsc.subcore_barrier()

    return msgpass(msg, ei2, zeros_h)


def _tc_norms_and_first(x, degs, corr, W1, S1w, S1b):
    n, d = x.shape
    f1 = W1.shape[1]

    def body(x_ref, degs_ref, corr_ref, w_ref, sw_ref, sb_ref, nsrc_ref,
             ndst_ref, msg_ref, s_ref):
        ds_ = degs_ref[0, 0] + degs_ref[0, 1] + 1.0 - corr_ref[...]
        dd_ = degs_ref[1, 0] + degs_ref[1, 1] + 1.0
        nsrc = lax.rsqrt(jnp.maximum(ds_, 1.0))
        ndst = lax.rsqrt(jnp.maximum(dd_, 1.0))
        nsrc_ref[...] = nsrc
        ndst_ref[...] = ndst
        xw = jnp.dot(x_ref[...], w_ref[...],
                     preferred_element_type=jnp.float32)
        msg_ref[0] = xw * nsrc
        s_ref[...] = jnp.dot(x_ref[...], sw_ref[...],
                             preferred_element_type=jnp.float32) + sb_ref[...]

    return pl.pallas_call(
        body,
        out_shape=[
            jax.ShapeDtypeStruct((n, 1), jnp.float32),
            jax.ShapeDtypeStruct((n, 1), jnp.float32),
            jax.ShapeDtypeStruct((1, n, f1), jnp.float32),
            jax.ShapeDtypeStruct((n, f1), jnp.float32),
        ],
    )(x, degs, corr, W1, S1w, S1b)


def _tc_combine(p, msg, s, b, g, be, ndst, c_chunks, fc):
    n = ndst.shape[0]

    def body(p_ref, msg_ref, s_ref, b_ref, g_ref, be_ref, ndst_ref, h_ref):
        agg = p_ref[0, 0] + p_ref[0, 1] + msg_ref[0]
        z = agg * ndst_ref[...] + b_ref[0]
        mu = jnp.mean(z, axis=0, keepdims=True)
        zc = z - mu
        var = jnp.mean(zc * zc, axis=0, keepdims=True)
        zn = zc * lax.rsqrt(var + 1e-5) * g_ref[0] + be_ref[0]
        h_ref[0] = jnp.maximum(zn + s_ref[0], 0.0)

    return pl.pallas_call(
        body,
        grid=(c_chunks,),
        in_specs=[
            pl.BlockSpec((1, 2, n, fc), lambda c: (c, 0, 0, 0)),
            pl.BlockSpec((1, n, fc), lambda c: (c, 0, 0)),
            pl.BlockSpec((1, n, fc), lambda c: (c, 0, 0)),
            pl.BlockSpec((1, 1, fc), lambda c: (c, 0, 0)),
            pl.BlockSpec((1, 1, fc), lambda c: (c, 0, 0)),
            pl.BlockSpec((1, 1, fc), lambda c: (c, 0, 0)),
            pl.BlockSpec((n, 1), lambda c: (0, 0)),
        ],
        out_specs=pl.BlockSpec((1, n, fc), lambda c: (c, 0, 0)),
        out_shape=jax.ShapeDtypeStruct((c_chunks, n, fc), jnp.float32),
    )(p, msg, s, b.reshape(c_chunks, 1, fc), g.reshape(c_chunks, 1, fc),
      be.reshape(c_chunks, 1, fc), ndst)


def _tc_transform(h, W, Sw, Sb, nsrc, cin, fcin, cout, fcout):
    n = nsrc.shape[0]
    din = cin * fcin

    def body(h_ref, w_ref, sw_ref, sb_ref, nsrc_ref, msg_ref, s_ref):
        accm = jnp.zeros((n, fcout), jnp.float32)
        accs = jnp.zeros((n, fcout), jnp.float32)
        for ci in range(cin):
            hc = h_ref[ci]
            accm += jnp.dot(hc, w_ref[pl.ds(ci * fcin, fcin), :],
                            preferred_element_type=jnp.float32)
            accs += jnp.dot(hc, sw_ref[pl.ds(ci * fcin, fcin), :],
                            preferred_element_type=jnp.float32)
        msg_ref[0] = accm * nsrc_ref[...]
        s_ref[0] = accs + sb_ref[0]

    return pl.pallas_call(
        body,
        grid=(cout,),
        in_specs=[
            pl.BlockSpec((cin, n, fcin), lambda c: (0, 0, 0)),
            pl.BlockSpec((din, fcout), lambda c: (0, c)),
            pl.BlockSpec((din, fcout), lambda c: (0, c)),
            pl.BlockSpec((1, 1, fcout), lambda c: (c, 0, 0)),
            pl.BlockSpec((n, 1), lambda c: (0, 0)),
        ],
        out_specs=[
            pl.BlockSpec((1, n, fcout), lambda c: (c, 0, 0)),
            pl.BlockSpec((1, n, fcout), lambda c: (c, 0, 0)),
        ],
        out_shape=[
            jax.ShapeDtypeStruct((cout, n, fcout), jnp.float32),
            jax.ShapeDtypeStruct((cout, n, fcout), jnp.float32),
        ],
    )(h, W, Sw, Sb.reshape(cout, 1, fcout), nsrc)


def _tc_msg_only(h, W, nsrc, cin, fcin, cout, fcout):
    n = nsrc.shape[0]
    din = cin * fcin

    def body(h_ref, w_ref, nsrc_ref, msg_ref):
        accm = jnp.zeros((n, fcout), jnp.float32)
        for ci in range(cin):
            accm += jnp.dot(h_ref[ci], w_ref[pl.ds(ci * fcin, fcin), :],
                            preferred_element_type=jnp.float32)
        msg_ref[0] = accm * nsrc_ref[...]

    return pl.pallas_call(
        body,
        grid=(cout,),
        in_specs=[
            pl.BlockSpec((cin, n, fcin), lambda c: (0, 0, 0)),
            pl.BlockSpec((din, fcout), lambda c: (0, c)),
            pl.BlockSpec((n, 1), lambda c: (0, 0)),
        ],
        out_specs=pl.BlockSpec((1, n, fcout), lambda c: (c, 0, 0)),
        out_shape=jax.ShapeDtypeStruct((cout, n, fcout), jnp.float32),
    )(h, W, nsrc)


def _tc_combine1_transform(p, msg, s, b, g, be, norms2, W2, S2w, S2b):
    n = norms2.shape[0]
    fo = W2.shape[1]

    def body(p_ref, msg_ref, s_ref, b_ref, g_ref, be_ref, norms_ref,
             w_ref, sw_ref, sb_ref, m_ref, so_ref):
        nsrc_v = norms_ref[:, 0:1]
        ndst_v = norms_ref[:, 1:2]
        agg = p_ref[0, 0] + p_ref[0, 1] + msg_ref[0]
        z = agg * ndst_v + b_ref[...]
        mu = jnp.mean(z, axis=0, keepdims=True)
        zc = z - mu
        var = jnp.mean(zc * zc, axis=0, keepdims=True)
        zn = zc * lax.rsqrt(var + 1e-5) * g_ref[...] + be_ref[...]
        h1 = jnp.maximum(zn + s_ref[...], 0.0)
        for co in range(fo // 128):
            cs = slice(co * 128, (co + 1) * 128)
            m_ref[co] = jnp.dot(h1, w_ref[:, cs],
                                preferred_element_type=jnp.float32
                                ) * nsrc_v
            so_ref[co] = jnp.dot(h1, sw_ref[:, cs],
                                 preferred_element_type=jnp.float32
                                 ) + sb_ref[cs]

    return pl.pallas_call(
        body,
        out_shape=[
            jax.ShapeDtypeStruct((fo // 128, n, 128), jnp.float32),
            jax.ShapeDtypeStruct((fo // 128, n, 128), jnp.float32),
        ],
    )(p, msg, s, b, g, be, norms2, W2, S2w, S2b)


def _tc_combine_heads(p, msg, s, b, g, be, ndst,
                      A1w, A1b, A2w, A2b, C1w, C1b, C2w, C2b,
                      c_chunks, fc):
    n = ndst.shape[0]

    def body(p_ref, msg_ref, s_ref, b_ref, g_ref, be_ref, ndst_ref,
             a1w_ref, a1b_ref, a2w_ref, a2b_ref,
             c1w_ref, c1b_ref, c2w_ref, c2b_ref,
             a_ref, c_ref, ta_ref, tc_ref):
        c = pl.program_id(0)
        agg = p_ref[0, 0] + p_ref[0, 1] + msg_ref[0]
        z = agg * ndst_ref[...] + b_ref[0]
        mu = jnp.mean(z, axis=0, keepdims=True)
        zc = z - mu
        var = jnp.mean(zc * zc, axis=0, keepdims=True)
        zn = zc * lax.rsqrt(var + 1e-5) * g_ref[0] + be_ref[0]
        hc = jnp.maximum(zn + s_ref[0], 0.0)
        pa = jnp.dot(hc, a1w_ref[0], preferred_element_type=jnp.float32)
        pc = jnp.dot(hc, c1w_ref[0], preferred_element_type=jnp.float32)

        @pl.when(c == 0)
        def _():
            ta_ref[...] = pa
            tc_ref[...] = pc

        @pl.when(c > 0)
        def _():
            ta_ref[...] += pa
            tc_ref[...] += pc

        @pl.when(c == c_chunks - 1)
        def _():
            ta = jnp.maximum(ta_ref[...] + a1b_ref[...], 0.0)
            tc = jnp.maximum(tc_ref[...] + c1b_ref[...], 0.0)
            a_ref[...] = jnp.dot(ta, a2w_ref[...],
                                 preferred_element_type=jnp.float32
                                 ) + a2b_ref[...]
            c_ref[...] = jnp.dot(tc, c2w_ref[...],
                                 preferred_element_type=jnp.float32
                                 ) + c2b_ref[...]

    return pl.pallas_call(
        body,
        grid=(c_chunks,),
        in_specs=[
            pl.BlockSpec((1, 2, n, fc), lambda c: (c, 0, 0, 0)),
            pl.BlockSpec((1, n, fc), lambda c: (c, 0, 0)),
            pl.BlockSpec((1, n, fc), lambda c: (c, 0, 0)),
            pl.BlockSpec((1, 1, fc), lambda c: (c, 0, 0)),
            pl.BlockSpec((1, 1, fc), lambda c: (c, 0, 0)),
            pl.BlockSpec((1, 1, fc), lambda c: (c, 0, 0)),
            pl.BlockSpec((n, 1), lambda c: (0, 0)),
            pl.BlockSpec((1, fc, 128), lambda c: (c, 0, 0)),
            pl.BlockSpec((128,), lambda c: (0,)),
            pl.BlockSpec((128, 1), lambda c: (0, 0)),
            pl.BlockSpec((1,), lambda c: (0,)),
            pl.BlockSpec((1, fc, 128), lambda c: (c, 0, 0)),
            pl.BlockSpec((128,), lambda c: (0,)),
            pl.BlockSpec((128, 1), lambda c: (0, 0)),
            pl.BlockSpec((1,), lambda c: (0,)),
        ],
        out_specs=[
            pl.BlockSpec((n, 1), lambda c: (0, 0)),
            pl.BlockSpec((n, 1), lambda c: (0, 0)),
        ],
        out_shape=[
            jax.ShapeDtypeStruct((n, 1), jnp.float32),
            jax.ShapeDtypeStruct((n, 1), jnp.float32),
        ],
        scratch_shapes=[
            pltpu.VMEM((n, 128), jnp.float32),
            pltpu.VMEM((n, 128), jnp.float32),
        ],
    )(p, msg, s, b.reshape(c_chunks, 1, fc), g.reshape(c_chunks, 1, fc),
      be.reshape(c_chunks, 1, fc), ndst,
      A1w.reshape(c_chunks, fc, 128), A1b, A2w, A2b,
      C1w.reshape(c_chunks, fc, 128), C1b, C2w, C2b)


def _tc_heads(h4, A1w, A1b, A2w, A2b, C1w, C1b, C2w, C2b, cin, fcin):
    n = h4.shape[1]

    def body(h_ref, a1w_ref, a1b_ref, a2w_ref, a2b_ref, c1w_ref, c1b_ref,
             c2w_ref, c2b_ref, a_ref, c_ref):
        ta = jnp.zeros((n, 128), jnp.float32)
        tc = jnp.zeros((n, 128), jnp.float32)
        for ci in range(cin):
            hc = h_ref[ci]
            ta += jnp.dot(hc, a1w_ref[pl.ds(ci * fcin, fcin), :],
                          preferred_element_type=jnp.float32)
            tc += jnp.dot(hc, c1w_ref[pl.ds(ci * fcin, fcin), :],
                          preferred_element_type=jnp.float32)
        ta = jnp.maximum(ta + a1b_ref[...], 0.0)
        tc = jnp.maximum(tc + c1b_ref[...], 0.0)
        a_ref[...] = jnp.dot(ta, a2w_ref[...],
                             preferred_element_type=jnp.float32) + a2b_ref[...]
        c_ref[...] = jnp.dot(tc, c2w_ref[...],
                             preferred_element_type=jnp.float32) + c2b_ref[...]

    return pl.pallas_call(
        body,
        out_shape=[
            jax.ShapeDtypeStruct((n, 1), jnp.float32),
            jax.ShapeDtypeStruct((n, 1), jnp.float32),
        ],
    )(h4, A1w, A1b, A2w, A2b, C1w, C1b, C2w, C2b)


def kernel(x, edge_index, W1, b1, W2, b2, W3, b3, W4, b4,
           g1, be1, g2, be2, g3, be3, g4, be4,
           S1w, S1b, S2w, S2b, S3w, S3b,
           A1w, A1b, A2w, A2b, C1w, C1b, C2w, C2b):
    n, d = x.shape
    e = edge_index.shape[1]
    e_pad = _NC * _NS * _NBT * _B
    npad = e_pad - e
    ntile = _NC * _NS
    dpt = npad // ntile
    ept = e // ntile
    didx = jnp.arange(npad, dtype=jnp.int32)
    dums = jnp.stack([(didx % n).reshape(ntile, dpt),
                      (n + didx % _PAD).reshape(ntile, dpt)])
    eip = jnp.concatenate([edge_index.reshape(2, ntile, ept), dums],
                          axis=2).reshape(2, -1, _B)
    pk = jax.lax.bitcast_convert_type(
        eip.astype(jnp.int16).reshape(2, -1, _B // 2, 2), jnp.int32)
    ei2 = pk.transpose(1, 0, 2).reshape(-1, 2 * (_B // 2))
    ones_h = jnp.ones((128,), jnp.float32)
    zeros1 = jnp.zeros((_round256(n + _PAD),), jnp.float32)
    nid = jnp.arange(n, dtype=jnp.int32)
    corr = (float(npad // n)
            + (nid < (npad % n)).astype(jnp.float32)).reshape(n, 1)

    degs = _sc_degrees(ei2, ones_h, zeros1, n=n)
    degs = degs[:, :, :n].reshape(2, 2, n, 1)

    pad64 = ((0, 0), (0, 64))
    W1p = jnp.pad(W1, pad64)
    S1wp = jnp.pad(S1w, pad64)
    S1bp = jnp.pad(S1b, (0, 64))
    b1p = jnp.pad(b1, (0, 64))
    g1p = jnp.pad(g1, (0, 64))
    be1p = jnp.pad(be1, (0, 64))
    W2p = jnp.pad(W2, ((0, 64), (0, 0)))
    S2wp = jnp.pad(S2w, ((0, 64), (0, 0)))

    nsrc, ndst, msg1, s1 = _tc_norms_and_first(x, degs, corr, W1p, S1wp, S1bp)

    z128 = jnp.zeros((n // _NS, 128), jnp.float32)
    p1 = _sc_msgpass(msg1, ei2, z128, n=n, fc=128, c_chunks=1)
    h1 = _tc_combine(p1, msg1, s1.reshape(1, n, 128), b1p, g1p, be1p,
                     ndst, 1, 128)

    msg2, s2 = _tc_transform(h1, W2p, S2wp, S2b, nsrc, 1, 128, 2, 128)
    p2 = _sc_msgpass(msg2, ei2, z128, n=n, fc=128, c_chunks=2)
    h2 = _tc_combine(p2, msg2, s2, b2, g2, be2, ndst, 2, 128)

    msg3, s3 = _tc_transform(h2, W3, S3w, S3b, nsrc, 2, 128, 4, 128)
    p3 = _sc_msgpass(msg3, ei2, z128, n=n, fc=128, c_chunks=4)
    h3 = _tc_combine(p3, msg3, s3, b3, g3, be3, ndst, 4, 128)

    msg4 = _tc_msg_only(h3, W4, nsrc, 4, 128, 4, 128)
    p4 = _sc_msgpass(msg4, ei2, z128, n=n, fc=128, c_chunks=4)
    h4 = _tc_combine(p4, msg4, h3, b4, g4, be4, ndst, 4, 128)
    active, consume = _tc_heads(h4, A1w, A1b, A2w, A2b, C1w, C1b, C2w, C2b,
                                4, 128)
    return (active, consume)


# speedup vs baseline: 1.1451x; 1.1451x over previous
import functools

import jax
import jax.numpy as jnp
from jax import lax
from jax.experimental import pallas as pl
from jax.experimental.pallas import tpu as pltpu
from jax.experimental.pallas import tpu_sc as plsc

_NC = 2
_NS = 16
_B = 128
_NBT = 80
_PAD = 512


def _sc_mesh():
    return plsc.VectorSubcoreMesh(core_axis_name="c", subcore_axis_name="s")


def _round256(n):
    return (n + 255) & ~255


@functools.partial(jax.jit, static_argnames=("n",))
def _sc_degrees(ei2, ones_h, zeros_h, n):
    nb_tot = ei2.shape[0]
    nb = nb_tot // (_NC * _NS)
    ra = (n // _NS) & ~7
    tail = n - ra * _NS

    @functools.partial(
        pl.kernel,
        mesh=_sc_mesh(),
        out_type=jax.ShapeDtypeStruct((2, _NC, _round256(n + _PAD)),
                                      jnp.float32),
        scratch_types=[
            pltpu.VMEM((nb, 128), jnp.int32),
            pltpu.VMEM((2, 128), jnp.int32),
            pltpu.VMEM((128,), jnp.float32),
            pltpu.VMEM_SHARED((_round256(n + _PAD),), jnp.float32),
            pltpu.VMEM_SHARED((_round256(n + _PAD),), jnp.float32),
        ],
    )
    def hist(ei_h, ones_hbm, zeros_hbm, out_h, ei_v, stg, ones_v,
             acc_s, acc_d):
        core = lax.axis_index("c")
        sub = lax.axis_index("s")
        row0 = (core * _NS + sub) * nb
        pltpu.sync_copy(ei_h.at[pl.ds(row0, nb)], ei_v)
        pltpu.sync_copy(ones_hbm, ones_v)

        @pl.when(sub == 0)
        def _():
            pltpu.sync_copy(zeros_hbm, acc_s)
            pltpu.sync_copy(zeros_hbm, acc_d)

        plsc.subcore_barrier()

        def body(j, carry):
            for d in range(2):
                for q in range(4):
                    v = ei_v[j, pl.ds(d * 64 + q * 16, 16)]
                    stg[d, pl.ds(q * 32, 16)] = v & 0xFFFF
                    stg[d, pl.ds(q * 32 + 16, 16)] = (
                        lax.shift_right_logical(v, 16))
            pltpu.sync_copy(ones_v, acc_s.at[stg.at[0]], add=True)
            pltpu.sync_copy(ones_v, acc_d.at[stg.at[1]], add=True)
            return carry

        lax.fori_loop(0, nb, body, 0)
        plsc.subcore_barrier()

        @pl.when(sub == 0)
        def _():
            pltpu.sync_copy(acc_s, out_h.at[0].at[core])
            pltpu.sync_copy(acc_d, out_h.at[1].at[core])

    return hist(ei2, ones_h, zeros_h)


@functools.partial(jax.jit, static_argnames=("n", "fc", "c_chunks"))
def _sc_msgpass(msg, ei2, zeros_h, n, fc, c_chunks):
    nb_tot = ei2.shape[0]
    nb = nb_tot // (_NC * _NS)
    ra = (n // _NS) & ~7
    tail = n - ra * _NS

    @functools.partial(
        pl.kernel,
        mesh=_sc_mesh(),
        out_type=jax.ShapeDtypeStruct((c_chunks, _NC, n, fc), jnp.float32),
        scratch_types=[
            pltpu.VMEM((nb, 128), jnp.int32),
            pltpu.VMEM((2, 2, _B), jnp.int32),
            pltpu.VMEM((2, _B, fc), jnp.float32),
            pltpu.VMEM_SHARED((n + _PAD, fc), jnp.float32),
            pltpu.SemaphoreType.DMA,
            pltpu.SemaphoreType.DMA,
        ],
    )
    def msgpass(msg_h, ei_h, zeros_hbm, out_h, ei_v, stg, rows, acc,
                sem0, sem1):
        core = lax.axis_index("c")
        sub = lax.axis_index("s")
        row0 = (core * _NS + sub) * nb
        pltpu.sync_copy(ei_h.at[pl.ds(row0, nb)], ei_v)
        sl = pl.ds(sub * ra, ra)
        tl = pl.ds(ra * _NS, tail)
        sems = (sem0, sem1)

        def unpack(j, b):
            for d in range(2):
                for q in range(4):
                    v = ei_v[j, pl.ds(d * 64 + q * 16, 16)]
                    stg[b, d, pl.ds(q * 32, 16)] = v & 0xFFFF
                    stg[b, d, pl.ds(q * 32 + 16, 16)] = (
                        lax.shift_right_logical(v, 16))

        for c in range(c_chunks):
            m = msg_h.at[c]

            @pl.when(core == 0)
            def _():
                pltpu.sync_copy(m.at[sl], acc.at[sl])

                @pl.when(sub == 0)
                def _():
                    pltpu.sync_copy(m.at[tl], acc.at[tl])

            @pl.when(core == 1)
            def _():
                pltpu.sync_copy(zeros_hbm.at[pl.ds(0, ra)], acc.at[sl])

                @pl.when(sub == 0)
                def _():
                    pltpu.sync_copy(zeros_hbm.at[pl.ds(0, tail)],
                                    acc.at[tl])

            plsc.subcore_barrier()

            unpack(0, 0)
            pltpu.async_copy(m.at[stg.at[0, 0]], rows.at[0], sems[0])

            def body(i, carry):
                j = 2 * i
                unpack(j + 1, 1)
                pltpu.async_copy(m.at[stg.at[1, 0]], rows.at[1], sems[1])
                pltpu.make_async_copy(m.at[stg.at[0, 0]], rows.at[0],
                                      sems[0]).wait()
                pltpu.sync_copy(rows.at[0], acc.at[stg.at[0, 1]], add=True)

                @pl.when(i < nb // 2 - 1)
                def _():
                    unpack(j + 2, 0)
                    pltpu.async_copy(m.at[stg.at[0, 0]], rows.at[0], sems[0])

                pltpu.make_async_copy(m.at[stg.at[1, 0]], rows.at[1],
                                      sems[1]).wait()
                pltpu.sync_copy(rows.at[1], acc.at[stg.at[1, 1]], add=True)
                return carry

            lax.fori_loop(0, nb // 2, body, 0)
            plsc.subcore_barrier()
            pltpu.sync_copy(acc.at[sl], out_h.at[c].at[core].at[sl])

            @pl.when(sub == 0)
            def _():
                pltpu.sync_copy(acc.at[tl], out_h.at[c].at[core].at[tl])

            plsc.subcore_barrier()

    return msgpass(msg, ei2, zeros_h)


def _tc_norms_and_first(x, degs, corr, W1, S1w, S1b):
    n, d = x.shape
    f1 = W1.shape[1]

    def body(x_ref, degs_ref, corr_ref, w_ref, sw_ref, sb_ref, nsrc_ref,
             ndst_ref, msg_ref, s_ref):
        ds_ = degs_ref[0, 0] + degs_ref[0, 1] + 1.0 - corr_ref[...]
        dd_ = degs_ref[1, 0] + degs_ref[1, 1] + 1.0
        nsrc = lax.rsqrt(jnp.maximum(ds_, 1.0))
        ndst = lax.rsqrt(jnp.maximum(dd_, 1.0))
        nsrc_ref[...] = nsrc
        ndst_ref[...] = ndst
        xw = jnp.dot(x_ref[...], w_ref[...],
                     preferred_element_type=jnp.float32)
        msg_ref[0] = xw * nsrc
        s_ref[...] = jnp.dot(x_ref[...], sw_ref[...],
                             preferred_element_type=jnp.float32) + sb_ref[...]

    return pl.pallas_call(
        body,
        out_shape=[
            jax.ShapeDtypeStruct((n, 1), jnp.float32),
            jax.ShapeDtypeStruct((n, 1), jnp.float32),
            jax.ShapeDtypeStruct((1, n, f1), jnp.float32),
            jax.ShapeDtypeStruct((n, f1), jnp.float32),
        ],
    )(x, degs, corr, W1, S1w, S1b)


def _tc_combine(p, s, b, g, be, ndst, c_chunks, fc):
    n = ndst.shape[0]

    def body(p_ref, s_ref, b_ref, g_ref, be_ref, ndst_ref, h_ref):
        agg = p_ref[0, 0] + p_ref[0, 1]
        z = agg * ndst_ref[...] + b_ref[0]
        mu = jnp.mean(z, axis=0, keepdims=True)
        zc = z - mu
        var = jnp.mean(zc * zc, axis=0, keepdims=True)
        zn = zc * lax.rsqrt(var + 1e-5) * g_ref[0] + be_ref[0]
        h_ref[0] = jnp.maximum(zn + s_ref[0], 0.0)

    return pl.pallas_call(
        body,
        grid=(c_chunks,),
        in_specs=[
            pl.BlockSpec((1, 2, n, fc), lambda c: (c, 0, 0, 0)),
            pl.BlockSpec((1, n, fc), lambda c: (c, 0, 0)),
            pl.BlockSpec((1, 1, fc), lambda c: (c, 0, 0)),
            pl.BlockSpec((1, 1, fc), lambda c: (c, 0, 0)),
            pl.BlockSpec((1, 1, fc), lambda c: (c, 0, 0)),
            pl.BlockSpec((n, 1), lambda c: (0, 0)),
        ],
        out_specs=pl.BlockSpec((1, n, fc), lambda c: (c, 0, 0)),
        out_shape=jax.ShapeDtypeStruct((c_chunks, n, fc), jnp.float32),
    )(p, s, b.reshape(c_chunks, 1, fc), g.reshape(c_chunks, 1, fc),
      be.reshape(c_chunks, 1, fc), ndst)


def _tc_transform(h, W, Sw, Sb, nsrc, cin, fcin, cout, fcout):
    n = nsrc.shape[0]
    din = cin * fcin

    def body(h_ref, w_ref, sw_ref, sb_ref, nsrc_ref, msg_ref, s_ref):
        accm = jnp.zeros((n, fcout), jnp.float32)
        accs = jnp.zeros((n, fcout), jnp.float32)
        for ci in range(cin):
            hc = h_ref[ci]
            accm += jnp.dot(hc, w_ref[pl.ds(ci * fcin, fcin), :],
                            preferred_element_type=jnp.float32)
            accs += jnp.dot(hc, sw_ref[pl.ds(ci * fcin, fcin), :],
                            preferred_element_type=jnp.float32)
        msg_ref[0] = accm * nsrc_ref[...]
        s_ref[0] = accs + sb_ref[0]

    return pl.pallas_call(
        body,
        grid=(cout,),
        in_specs=[
            pl.BlockSpec((cin, n, fcin), lambda c: (0, 0, 0)),
            pl.BlockSpec((din, fcout), lambda c: (0, c)),
            pl.BlockSpec((din, fcout), lambda c: (0, c)),
            pl.BlockSpec((1, 1, fcout), lambda c: (c, 0, 0)),
            pl.BlockSpec((n, 1), lambda c: (0, 0)),
        ],
        out_specs=[
            pl.BlockSpec((1, n, fcout), lambda c: (c, 0, 0)),
            pl.BlockSpec((1, n, fcout), lambda c: (c, 0, 0)),
        ],
        out_shape=[
            jax.ShapeDtypeStruct((cout, n, fcout), jnp.float32),
            jax.ShapeDtypeStruct((cout, n, fcout), jnp.float32),
        ],
    )(h, W, Sw, Sb.reshape(cout, 1, fcout), nsrc)


def _tc_msg_only(h, W, nsrc, cin, fcin, cout, fcout):
    n = nsrc.shape[0]
    din = cin * fcin

    def body(h_ref, w_ref, nsrc_ref, msg_ref):
        accm = jnp.zeros((n, fcout), jnp.float32)
        for ci in range(cin):
            accm += jnp.dot(h_ref[ci], w_ref[pl.ds(ci * fcin, fcin), :],
                            preferred_element_type=jnp.float32)
        msg_ref[0] = accm * nsrc_ref[...]

    return pl.pallas_call(
        body,
        grid=(cout,),
        in_specs=[
            pl.BlockSpec((cin, n, fcin), lambda c: (0, 0, 0)),
            pl.BlockSpec((din, fcout), lambda c: (0, c)),
            pl.BlockSpec((n, 1), lambda c: (0, 0)),
        ],
        out_specs=pl.BlockSpec((1, n, fcout), lambda c: (c, 0, 0)),
        out_shape=jax.ShapeDtypeStruct((cout, n, fcout), jnp.float32),
    )(h, W, nsrc)


def _tc_heads(h4, A1w, A1b, A2w, A2b, C1w, C1b, C2w, C2b, cin, fcin):
    n = h4.shape[1]

    def body(h_ref, a1w_ref, a1b_ref, a2w_ref, a2b_ref, c1w_ref, c1b_ref,
             c2w_ref, c2b_ref, a_ref, c_ref):
        ta = jnp.zeros((n, 128), jnp.float32)
        tc = jnp.zeros((n, 128), jnp.float32)
        for ci in range(cin):
            hc = h_ref[ci]
            ta += jnp.dot(hc, a1w_ref[pl.ds(ci * fcin, fcin), :],
                          preferred_element_type=jnp.float32)
            tc += jnp.dot(hc, c1w_ref[pl.ds(ci * fcin, fcin), :],
                          preferred_element_type=jnp.float32)
        ta = jnp.maximum(ta + a1b_ref[...], 0.0)
        tc = jnp.maximum(tc + c1b_ref[...], 0.0)
        a_ref[...] = jnp.dot(ta, a2w_ref[...],
                             preferred_element_type=jnp.float32) + a2b_ref[...]
        c_ref[...] = jnp.dot(tc, c2w_ref[...],
                             preferred_element_type=jnp.float32) + c2b_ref[...]

    return pl.pallas_call(
        body,
        out_shape=[
            jax.ShapeDtypeStruct((n, 1), jnp.float32),
            jax.ShapeDtypeStruct((n, 1), jnp.float32),
        ],
    )(h4, A1w, A1b, A2w, A2b, C1w, C1b, C2w, C2b)


def kernel(x, edge_index, W1, b1, W2, b2, W3, b3, W4, b4,
           g1, be1, g2, be2, g3, be3, g4, be4,
           S1w, S1b, S2w, S2b, S3w, S3b,
           A1w, A1b, A2w, A2b, C1w, C1b, C2w, C2b):
    n, d = x.shape
    e = edge_index.shape[1]
    e_pad = _NC * _NS * _NBT * _B
    npad = e_pad - e
    ntile = _NC * _NS
    dpt = npad // ntile
    ept = e // ntile
    didx = jnp.arange(npad, dtype=jnp.int32)
    dsrc = (didx % n).reshape(ntile, dpt)
    ddst = (n + didx % _PAD).reshape(ntile, dpt)
    src_p = jnp.concatenate([edge_index[0].reshape(ntile, ept), dsrc],
                            axis=1).reshape(-1, _B)
    dst_p = jnp.concatenate([edge_index[1].reshape(ntile, ept), ddst],
                            axis=1).reshape(-1, _B)
    ps = src_p[:, 0::2] | (src_p[:, 1::2] << 16)
    pd = dst_p[:, 0::2] | (dst_p[:, 1::2] << 16)
    ei2 = jnp.concatenate([ps, pd], axis=1)
    ones_h = jnp.ones((128,), jnp.float32)
    zeros1 = jnp.zeros((_round256(n + _PAD),), jnp.float32)
    nid = jnp.arange(n, dtype=jnp.int32)
    corr = (float(npad // n)
            + (nid < (npad % n)).astype(jnp.float32)).reshape(n, 1)

    degs = _sc_degrees(ei2, ones_h, zeros1, n=n)
    degs = degs[:, :, :n].reshape(2, 2, n, 1)

    pad64 = ((0, 0), (0, 64))
    W1p = jnp.pad(W1, pad64)
    S1wp = jnp.pad(S1w, pad64)
    S1bp = jnp.pad(S1b, (0, 64))
    b1p = jnp.pad(b1, (0, 64))
    g1p = jnp.pad(g1, (0, 64))
    be1p = jnp.pad(be1, (0, 64))
    W2p = jnp.pad(W2, ((0, 64), (0, 0)))
    S2wp = jnp.pad(S2w, ((0, 64), (0, 0)))

    nsrc, ndst, msg1, s1 = _tc_norms_and_first(x, degs, corr, W1p, S1wp, S1bp)

    z128 = jnp.zeros((n // _NS, 128), jnp.float32)
    p1 = _sc_msgpass(msg1, ei2, z128, n=n, fc=128, c_chunks=1)
    h1 = _tc_combine(p1, s1.reshape(1, n, 128), b1p, g1p, be1p,
                     ndst, 1, 128)

    msg2, s2 = _tc_transform(h1, W2p, S2wp, S2b, nsrc, 1, 128, 2, 128)
    p2 = _sc_msgpass(msg2, ei2, z128, n=n, fc=128, c_chunks=2)
    h2 = _tc_combine(p2, s2, b2, g2, be2, ndst, 2, 128)

    msg3, s3 = _tc_transform(h2, W3, S3w, S3b, nsrc, 2, 128, 4, 128)
    p3 = _sc_msgpass(msg3, ei2, z128, n=n, fc=128, c_chunks=4)
    h3 = _tc_combine(p3, s3, b3, g3, be3, ndst, 4, 128)

    msg4 = _tc_msg_only(h3, W4, nsrc, 4, 128, 4, 128)
    p4 = _sc_msgpass(msg4, ei2, z128, n=n, fc=128, c_chunks=4)
    h4 = _tc_combine(p4, h3, b4, g4, be4, ndst, 4, 128)

    active, consume = _tc_heads(h4, A1w, A1b, A2w, A2b, C1w, C1b, C2w, C2b,
                                4, 128)
    return (active, consume)
